# baseline (speedup 1.0000x reference)
import numpy as np
import jax
import jax.numpy as jnp
from jax.experimental import pallas as pl
from jax.experimental.pallas import tpu as pltpu
from jax.experimental.pallas import tpu_sc as plsc

_B, _C, _H, _W = 2, 96, 64, 64
_P = _H * _W
_K = 16
_OUT_C = 96
_BLK = 256


def _segment_onehots():
    g = int(np.floor(np.sqrt(_K)))
    rows = (np.arange(_H) * g) // _H
    cols = (np.arange(_W) * g) // _W
    seg = (rows[:, None] * g + cols[None, :]).reshape(-1)
    sm = np.zeros((_P, _K), np.float32)
    sm[np.arange(_P), seg] = 1.0
    sg = sm.T.copy()
    return sg


_SG_NP = _segment_onehots()

_VECTOR_MESH = plsc.VectorSubcoreMesh(
    core_axis_name="core", subcore_axis_name="subcore")


def _block_sums_sc(x_blocks):
    n = x_blocks.shape[0]

    @pl.kernel(out_type=jax.ShapeDtypeStruct((n, _C), jnp.float32),
               mesh=_VECTOR_MESH)
    def sc_kernel(x_hbm, o_hbm):
        def body(in_vmem, out_vmem):
            for c in range(0, _C, 16):
                out_vmem.at[0, pl.ds(c, 16)][...] = jnp.zeros((16,), jnp.float32)

            @pl.loop(0, _BLK)
            def _(r):
                for c in range(0, _C, 16):
                    out_vmem.at[0, pl.ds(c, 16)][...] = (
                        out_vmem.at[0, pl.ds(c, 16)][...]
                        + in_vmem.at[0, r, pl.ds(c, 16)][...])

        pltpu.emit_pipeline(
            body,
            grid=(n,),
            in_specs=[pl.BlockSpec((1, _BLK, _C), lambda i: (i, 0, 0))],
            out_specs=[pl.BlockSpec((1, _C), lambda i: (i, 0))],
            core_axis_name=("core", "subcore"),
            dimension_semantics=(pltpu.PARALLEL,),
        )(x_hbm, o_hbm)

    return sc_kernel(x_blocks)


def _dgfu_tc_kernel(x_ref, sums_ref, w_ref, taps_ref, sg_ref, out_ref):
    x = x_ref[0]
    means_kc = sums_ref[0] * (1.0 / _BLK)

    mw = jnp.dot(means_kc, w_ref[...], preferred_element_type=jnp.float32)
    sq = jnp.sum(mw * mw, axis=1, keepdims=True)
    gram = jnp.dot(mw, mw.T, preferred_element_type=jnp.float32)
    quad = sq + sq.T - 2.0 * gram
    row = jax.lax.broadcasted_iota(jnp.int32, (_K, _K), 0)
    col = jax.lax.broadcasted_iota(jnp.int32, (_K, _K), 1)
    adj = jnp.exp(-quad) * (row != col).astype(jnp.float32)

    am_kc = jnp.dot(adj, means_kc, preferred_element_type=jnp.float32)
    gathered = jnp.dot(am_kc.T, sg_ref[...], preferred_element_type=jnp.float32)
    feat = (x + gathered).astype(jnp.bfloat16)

    pid = jax.lax.broadcasted_iota(jnp.int32, (1, _P), 1)
    wcol = pid & (_W - 1)
    hrow = pid >> 6
    m_top = (hrow != 0).astype(jnp.bfloat16)
    m_bot = (hrow != (_H - 1)).astype(jnp.bfloat16)
    m_lft = (wcol != 0).astype(jnp.float32)
    m_rgt = (wcol != (_W - 1)).astype(jnp.float32)
    taps_bf = taps_ref[...].astype(jnp.bfloat16)

    v = {
        -1: pltpu.roll(feat, _W, axis=1) * m_top,
        0: feat,
        1: pltpu.roll(feat, _P - _W, axis=1) * m_bot,
    }
    acc = jnp.zeros((_OUT_C, _P), jnp.float32)
    for dx in (-1, 0, 1):
        g = jnp.zeros((_OUT_C, _P), jnp.float32)
        for dy in (-1, 0, 1):
            t = (dy + 1) * 3 + (dx + 1)
            g = g + jnp.dot(taps_bf[t], v[dy],
                            preferred_element_type=jnp.float32)
        if dx == -1:
            acc = acc + pltpu.roll(g, 1, axis=1) * m_lft
        elif dx == 1:
            acc = acc + pltpu.roll(g, _P - 1, axis=1) * m_rgt
        else:
            acc = acc + g
    out_ref[0] = acc


def kernel(x, W, conv_w):
    Bn, Cn, Hn, Wd = x.shape
    x_flat = x.reshape(Bn, Cn, Hn * Wd)
    x_blocks = (x.reshape(Bn, Cn, 4, 16, 4, 16)
                .transpose(0, 2, 4, 3, 5, 1)
                .reshape(Bn * _K, _BLK, Cn))
    sums = _block_sums_sc(x_blocks).reshape(Bn, _K, _C)

    taps = conv_w.transpose(2, 3, 0, 1).reshape(9, _OUT_C, _C)
    sg = jnp.asarray(_SG_NP)

    out_flat = pl.pallas_call(
        _dgfu_tc_kernel,
        grid=(Bn,),
        in_specs=[
            pl.BlockSpec((1, _C, _P), lambda b: (b, 0, 0)),
            pl.BlockSpec((1, _K, _C), lambda b: (b, 0, 0)),
            pl.BlockSpec((_C, _C), lambda b: (0, 0)),
            pl.BlockSpec((9, _OUT_C, _C), lambda b: (0, 0, 0)),
            pl.BlockSpec((_K, _P), lambda b: (0, 0)),
        ],
        out_specs=pl.BlockSpec((1, _OUT_C, _P), lambda b: (b, 0, 0)),
        out_shape=jax.ShapeDtypeStruct((Bn, _OUT_C, _P), jnp.float32),
        compiler_params=pltpu.CompilerParams(
            dimension_semantics=("parallel",),
        ),
    )(x_flat, sums, W, taps, sg)
    return out_flat.reshape(Bn, _OUT_C, Hn, Wd)


# speedup vs baseline: 2.0953x; 2.0953x over previous
import numpy as np
import jax
import jax.numpy as jnp
from jax.experimental import pallas as pl
from jax.experimental.pallas import tpu as pltpu

_B, _C, _H, _W = 2, 96, 64, 64
_P = _H * _W
_K = 16
_OUT_C = 96


def _segment_onehots():
    g = int(np.floor(np.sqrt(_K)))
    rows = (np.arange(_H) * g) // _H
    cols = (np.arange(_W) * g) // _W
    seg = (rows[:, None] * g + cols[None, :]).reshape(-1)
    sm = np.zeros((_P, _K), np.float32)
    sm[np.arange(_P), seg] = 1.0
    counts = sm.sum(axis=0)
    denom = counts + (counts == 0)
    sm_mean = sm / denom[None, :]
    sg = sm.T.copy()
    return sm_mean, sg


_SM_NP, _SG_NP = _segment_onehots()


def _dgfu_kernel(x_ref, m_ref, taps_ref, sm_ref, sg_ref, out_ref):
    x = x_ref[0].astype(jnp.bfloat16)

    means_ck = jnp.dot(x, sm_ref[...], preferred_element_type=jnp.float32)

    q1 = jax.lax.dot_general(means_ck, m_ref[...], (((0,), (0,)), ((), ())),
                             preferred_element_type=jnp.float32)
    gram = jnp.dot(q1, means_ck, preferred_element_type=jnp.float32)
    row = jax.lax.broadcasted_iota(jnp.int32, (_K, _K), 0)
    col = jax.lax.broadcasted_iota(jnp.int32, (_K, _K), 1)
    eye = (row == col).astype(jnp.float32)
    diag = gram * eye
    sq_c = jnp.sum(diag, axis=1, keepdims=True)
    sq_r = jnp.sum(diag, axis=0, keepdims=True)
    quad = sq_c + sq_r - 2.0 * gram
    adj = jnp.exp(-quad) * (1.0 - eye)

    am_ck = jnp.dot(means_ck, adj, preferred_element_type=jnp.float32)
    gathered = jnp.dot(am_ck, sg_ref[...], preferred_element_type=jnp.float32)
    feat = x + gathered.astype(jnp.bfloat16)

    pid = jax.lax.broadcasted_iota(jnp.int32, (1, _P), 1)
    wcol = pid & (_W - 1)
    hrow = pid >> 6
    m_top = (hrow != 0).astype(jnp.bfloat16)
    m_bot = (hrow != (_H - 1)).astype(jnp.bfloat16)
    m_lft = (wcol != 0).astype(jnp.float32)
    m_rgt = (wcol != (_W - 1)).astype(jnp.float32)
    taps_bf = taps_ref[...].astype(jnp.bfloat16)

    v = {
        -1: pltpu.roll(feat, _W, axis=1) * m_top,
        0: feat,
        1: pltpu.roll(feat, _P - _W, axis=1) * m_bot,
    }
    acc = jnp.zeros((_OUT_C, _P), jnp.float32)
    for dx in (-1, 0, 1):
        g = jnp.zeros((_OUT_C, _P), jnp.float32)
        for dy in (-1, 0, 1):
            t = (dy + 1) * 3 + (dx + 1)
            g = g + jnp.dot(taps_bf[t], v[dy],
                            preferred_element_type=jnp.float32)
        if dx == -1:
            acc = acc + pltpu.roll(g, 1, axis=1) * m_lft
        elif dx == 1:
            acc = acc + pltpu.roll(g, _P - 1, axis=1) * m_rgt
        else:
            acc = acc + g
    out_ref[0] = acc


def kernel(x, W, conv_w):
    Bn, Cn, Hn, Wd = x.shape
    x_flat = x.reshape(Bn, Cn, Hn * Wd)
    taps = conv_w.transpose(2, 3, 0, 1).reshape(9, _OUT_C, _C)
    M = jnp.dot(W, W.T)
    sm = jnp.asarray(_SM_NP, dtype=jnp.bfloat16)
    sg = jnp.asarray(_SG_NP)

    out_flat = pl.pallas_call(
        _dgfu_kernel,
        grid=(Bn,),
        in_specs=[
            pl.BlockSpec((1, _C, _P), lambda b: (b, 0, 0)),
            pl.BlockSpec((_C, _C), lambda b: (0, 0)),
            pl.BlockSpec((9, _OUT_C, _C), lambda b: (0, 0, 0)),
            pl.BlockSpec((_P, _K), lambda b: (0, 0)),
            pl.BlockSpec((_K, _P), lambda b: (0, 0)),
        ],
        out_specs=pl.BlockSpec((1, _OUT_C, _P), lambda b: (b, 0, 0)),
        out_shape=jax.ShapeDtypeStruct((Bn, _OUT_C, _P), jnp.float32),
        compiler_params=pltpu.CompilerParams(
            dimension_semantics=("parallel",),
        ),
    )(x_flat, M, taps, sm, sg)
    return out_flat.reshape(Bn, _OUT_C, Hn, Wd)


# speedup vs baseline: 2.1285x; 1.0158x over previous
import numpy as np
import jax
import jax.numpy as jnp
from jax.experimental import pallas as pl
from jax.experimental.pallas import tpu as pltpu

_B, _C, _H, _W = 2, 96, 64, 64
_P = _H * _W
_K = 16
_OUT_C = 96


def _segment_onehots():
    g = int(np.floor(np.sqrt(_K)))
    rows = (np.arange(_H) * g) // _H
    cols = (np.arange(_W) * g) // _W
    seg = (rows[:, None] * g + cols[None, :]).reshape(-1)
    sm = np.zeros((_P, _K), np.float32)
    sm[np.arange(_P), seg] = 1.0
    counts = sm.sum(axis=0)
    denom = counts + (counts == 0)
    sm_mean = sm / denom[None, :]
    sg = sm.T.copy()
    return sm_mean, sg


_SM_NP, _SG_NP = _segment_onehots()


def _dgfu_kernel(x_ref, w_ref, taps_ref, sm_ref, sg_ref, out_ref):
    x = x_ref[0]

    means_ck = jnp.dot(x, sm_ref[...], preferred_element_type=jnp.float32)
    means_kc = means_ck.T

    mw = jnp.dot(means_kc, w_ref[...], preferred_element_type=jnp.float32)
    sq = jnp.sum(mw * mw, axis=1, keepdims=True)
    gram = jnp.dot(mw, mw.T, preferred_element_type=jnp.float32)
    quad = sq + sq.T - 2.0 * gram
    row = jax.lax.broadcasted_iota(jnp.int32, (_K, _K), 0)
    col = jax.lax.broadcasted_iota(jnp.int32, (_K, _K), 1)
    adj = jnp.exp(-quad) * (row != col).astype(jnp.float32)

    am_kc = jnp.dot(adj, means_kc, preferred_element_type=jnp.float32)
    gathered = jnp.dot(am_kc.T, sg_ref[...], preferred_element_type=jnp.float32)
    feat = (x + gathered).astype(jnp.bfloat16)

    pid = jax.lax.broadcasted_iota(jnp.int32, (1, _P), 1)
    wcol = pid & (_W - 1)
    hrow = pid >> 6
    m_top = (hrow != 0).astype(jnp.bfloat16)
    m_bot = (hrow != (_H - 1)).astype(jnp.bfloat16)
    m_lft = (wcol != 0).astype(jnp.float32)
    m_rgt = (wcol != (_W - 1)).astype(jnp.float32)
    taps_bf = taps_ref[...].astype(jnp.bfloat16)

    v = {
        -1: pltpu.roll(feat, _W, axis=1) * m_top,
        0: feat,
        1: pltpu.roll(feat, _P - _W, axis=1) * m_bot,
    }
    acc = jnp.zeros((_OUT_C, _P), jnp.float32)
    for dx in (-1, 0, 1):
        g = jnp.zeros((_OUT_C, _P), jnp.float32)
        for dy in (-1, 0, 1):
            t = (dy + 1) * 3 + (dx + 1)
            g = g + jnp.dot(taps_bf[t], v[dy],
                            preferred_element_type=jnp.float32)
        if dx == -1:
            acc = acc + pltpu.roll(g, 1, axis=1) * m_lft
        elif dx == 1:
            acc = acc + pltpu.roll(g, _P - 1, axis=1) * m_rgt
        else:
            acc = acc + g
    out_ref[0] = acc


def kernel(x, W, conv_w):
    Bn, Cn, Hn, Wd = x.shape
    x_flat = x.reshape(Bn, Cn, Hn * Wd)
    taps = conv_w.transpose(2, 3, 0, 1).reshape(9, _OUT_C, _C)
    sm = jnp.asarray(_SM_NP)
    sg = jnp.asarray(_SG_NP)

    out_flat = pl.pallas_call(
        _dgfu_kernel,
        grid=(Bn,),
        in_specs=[
            pl.BlockSpec((1, _C, _P), lambda b: (b, 0, 0)),
            pl.BlockSpec((_C, _C), lambda b: (0, 0)),
            pl.BlockSpec((9, _OUT_C, _C), lambda b: (0, 0, 0)),
            pl.BlockSpec((_P, _K), lambda b: (0, 0)),
            pl.BlockSpec((_K, _P), lambda b: (0, 0)),
        ],
        out_specs=pl.BlockSpec((1, _OUT_C, _P), lambda b: (b, 0, 0)),
        out_shape=jax.ShapeDtypeStruct((Bn, _OUT_C, _P), jnp.float32),
        compiler_params=pltpu.CompilerParams(
            dimension_semantics=("parallel",),
        ),
    )(x_flat, W, taps, sm, sg)
    return out_flat.reshape(Bn, _OUT_C, Hn, Wd)
